# baseline (speedup 1.0000x reference)
import functools

import jax
import jax.numpy as jnp
from jax import lax
from jax.experimental import pallas as pl
from jax.experimental.pallas import tpu as pltpu
from jax.experimental.pallas import tpu_sc as plsc

DIM = 128
LANES = 16
CHUNK = 128
NBUF = 4


def _emb_body(t_per_w, seq_len, batch, num_cores,
              idx_hbm, tok_hbm, pos_hbm, out_hbm,
              idx_v, rows0, rows1, rows2, rows3, pos_v,
              sem_g, sem_o, sem_p):
  cid = lax.axis_index("c")
  sid = lax.axis_index("s")
  wid = sid * num_cores + cid
  t0 = wid * t_per_w
  n_rows = t_per_w * batch
  n_chunks = n_rows // CHUNK
  per_b = t_per_w // CHUNK
  n_groups = n_chunks // NBUF

  row_bufs = (rows0, rows1, rows2, rows3)

  def idx_slice(k):
    return idx_v.at[pl.ds(k * CHUNK, CHUNK)]

  def gather_cp(k, buf):
    return pltpu.make_async_copy(tok_hbm.at[idx_slice(k)], buf, sem_g)

  def write_cp(k, buf):
    b_row = k // per_b
    h = lax.rem(k, per_b)
    return pltpu.make_async_copy(
        buf, out_hbm.at[b_row, pl.ds(t0 + h * CHUNK, CHUNK)], sem_o)

  def add_pos(k, rows_v):
    p0 = lax.rem(k, per_b) * CHUNK

    def add_row(i, c):
      r = i * 2
      for u in range(2):
        for j in range(DIM // LANES):
          s = pl.ds(j * LANES, LANES)
          plsc.addupdate(rows_v.at[r + u, s], pos_v[p0 + r + u, s])
      return c

    lax.fori_loop(0, CHUNK // 2, add_row, 0)

  idx_cps = []
  for b in range(batch):
    cp = pltpu.make_async_copy(
        idx_hbm.at[b, pl.ds(t0, t_per_w)],
        idx_v.at[pl.ds(b * t_per_w, t_per_w)],
        sem_p,
    )
    cp.start()
    idx_cps.append(cp)
  for cp in idx_cps:
    cp.wait()

  for k in range(NBUF - 1):
    gather_cp(k, row_bufs[k]).start()
  pos_cp = pltpu.make_async_copy(pos_hbm.at[pl.ds(t0, t_per_w)], pos_v, sem_p)
  pos_cp.start()

  def group(g, carry):
    for b in range(NBUF):
      k = g * NBUF + b
      prev_buf = row_bufs[(b - 1) % NBUF]
      if b == 0:
        @pl.when(g >= 1)
        def _():
          write_cp(k - 1, prev_buf).wait()
        gather_cp(k + NBUF - 1, prev_buf).start()
      else:
        write_cp(k - 1, prev_buf).wait()

        @pl.when(k + NBUF - 1 < n_chunks)
        def _():
          gather_cp(k + NBUF - 1, prev_buf).start()
      gather_cp(k, row_bufs[b]).wait()
      if b == 0:
        @pl.when(g == 0)
        def _():
          pos_cp.wait()
      add_pos(k, row_bufs[b])
      write_cp(k, row_bufs[b]).start()
    return carry

  lax.fori_loop(0, n_groups, group, 0)
  write_cp(n_chunks - 1, row_bufs[NBUF - 1]).wait()


def kernel(idx, token_emb, pos_emb):
  B, T = idx.shape
  if idx.dtype != jnp.int32:
    idx = idx.astype(jnp.int32)
  info = plsc.get_sparse_core_info()
  num_workers = info.num_cores * info.num_subcores
  t_per_w = T // num_workers

  mesh = plsc.VectorSubcoreMesh(core_axis_name="c", subcore_axis_name="s")
  run = functools.partial(
      pl.kernel,
      mesh=mesh,
      out_type=jax.ShapeDtypeStruct((B, T, DIM), jnp.float32),
      scratch_types=[
          pltpu.VMEM((t_per_w * B,), jnp.int32),
          pltpu.VMEM((CHUNK, DIM), jnp.float32),
          pltpu.VMEM((CHUNK, DIM), jnp.float32),
          pltpu.VMEM((CHUNK, DIM), jnp.float32),
          pltpu.VMEM((CHUNK, DIM), jnp.float32),
          pltpu.VMEM((t_per_w, DIM), jnp.float32),
          pltpu.SemaphoreType.DMA,
          pltpu.SemaphoreType.DMA,
          pltpu.SemaphoreType.DMA,
      ],
  )(functools.partial(_emb_body, t_per_w, T, B, info.num_cores))

  return run(idx, token_emb, pos_emb)
